# baseline (speedup 1.0000x reference)
import jax
import jax.numpy as jnp
from jax.experimental import pallas as pl
from jax.experimental.pallas import tpu as pltpu

_TN = 10240


def _ln(t, g, b):
    mu = jnp.mean(t, axis=-1, keepdims=True)
    va = jnp.mean((t - mu) ** 2, axis=-1, keepdims=True)
    return (t - mu) * jax.lax.rsqrt(va + 1e-5) * g + b


def _dot(a, b, dims):
    return jax.lax.dot_general(a, b, (dims, ((), ())),
                               preferred_element_type=jnp.float32)


def _make_fused(T, tn):
    def fused(x_ref, e_ref, ev_ref, mrow_ref, mcol_ref,
              w1_ref, b1_ref, g1_ref, bb1_ref,
              w2_ref, b2_ref, g2_ref, bb2_ref,
              wq_ref, bq_ref, wk_ref, bk_ref, wv_ref, bv_ref,
              wo_ref, bo_ref, wf1_ref, bf1_ref, wf2_ref, bf2_ref,
              wp_ref, bp_ref, gp_ref, bbp_ref,
              out_ref, acc_ref, m_ref, ecache_ref):
        i = pl.program_id(0)

        @pl.when(i < T)
        def _():
            e_tile = e_ref[...]
            part = _dot(e_tile, x_ref[...], ((0,), (0,)))
            ecache_ref[:, pl.ds(i * tn, tn)] = e_tile.T

            @pl.when(i == 0)
            def _():
                acc_ref[...] = part

            @pl.when(i > 0)
            def _():
                acc_ref[...] = acc_ref[...] + part

        @pl.when(i == T - 1)
        def _():
            h = ev_ref[...] * w1_ref[...] + b1_ref[...]
            h = _ln(h, g1_ref[...], bb1_ref[...])
            h = jnp.maximum(h, 0.0)
            h = _dot(h, w2_ref[...], ((1,), (1,))) + b2_ref[...]
            h = _ln(h, g2_ref[...], bb2_ref[...])
            q = _dot(h, wq_ref[...], ((1,), (1,))) + bq_ref[...]
            k = _dot(h, wk_ref[...], ((1,), (1,))) + bk_ref[...]
            v = _dot(h, wv_ref[...], ((1,), (1,))) + bv_ref[...]
            mrow = mrow_ref[...]
            ctx_parts = []
            for hh in range(4):
                sl = slice(8 * hh, 8 * hh + 8)
                qh, kh, vh = q[:, sl], k[:, sl], v[:, sl]
                s = _dot(qh, kh, ((1,), (1,))) * (1.0 / jnp.sqrt(8.0))
                s = jnp.where(mrow == 0.0, -1e9, s)
                s = s - jnp.max(s, axis=-1, keepdims=True)
                e = jnp.exp(s)
                a = e / jnp.sum(e, axis=-1, keepdims=True)
                ctx_parts.append(_dot(a, vh, ((1,), (0,))))
            ctx = jnp.concatenate(ctx_parts, axis=1)
            ctx = _dot(ctx, wo_ref[...], ((1,), (1,))) + bo_ref[...]
            g = jnp.maximum(
                _dot(ctx, wf1_ref[...], ((1,), (1,))) + bf1_ref[...], 0.0)
            f = jnp.tanh(jnp.sum(g * wf2_ref[...], axis=1, keepdims=True)
                         + bf2_ref[...])
            f = f * mcol_ref[...]
            m_ref[...] = _dot(f * acc_ref[...], wp_ref[...], ((1,), (1,)))

        @pl.when(i >= T)
        def _():
            et_tile = ecache_ref[:, pl.ds((i - T) * tn, tn)]
            y = _dot(et_tile, m_ref[...], ((0,), (0,))) + bp_ref[...]
            out_ref[...] = _ln(y, gp_ref[...], bbp_ref[...])

    return fused


def kernel(x, eigenvectors, eigenvalues, W1, b1, g1, bb1, W2, b2, g2, bb2,
           Wq, bq, Wk, bk, Wv, bv, Wo, bo, Wf1, bf1, Wf2, bf2,
           Wp, bp, gp, bbp, eig_mask, batch):
    N, D = x.shape
    K = eigenvalues.shape[0]
    OD = Wp.shape[0]
    tn = _TN
    npad = (-N) % tn
    if npad:
        x = jnp.pad(x, ((0, npad), (0, 0)))
        eigenvectors = jnp.pad(eigenvectors, ((0, npad), (0, 0)))
    Np = N + npad
    T = Np // tn

    row = lambda a: a.reshape(1, -1).astype(jnp.float32)
    full = lambda shp: pl.BlockSpec(shp, lambda i: (0, 0))

    smalls = (
        eigenvalues.reshape(K, 1),
        row(eig_mask), eig_mask.astype(jnp.float32).reshape(K, 1),
        row(W1), row(b1), row(g1), row(bb1),
        W2, row(b2), row(g2), row(bb2),
        Wq, row(bq), Wk, row(bk), Wv, row(bv),
        Wo, row(bo), Wf1, row(bf1), row(Wf2), row(bf2),
        Wp, row(bp), row(gp), row(bbp),
    )
    small_specs = [full(a.shape) for a in smalls]

    out = pl.pallas_call(
        _make_fused(T, tn),
        grid=(2 * T,),
        in_specs=[pl.BlockSpec((tn, D), lambda i: (jnp.minimum(i, T - 1), 0)),
                  pl.BlockSpec((tn, K), lambda i: (jnp.minimum(i, T - 1), 0))]
                 + small_specs,
        out_specs=pl.BlockSpec((tn, OD), lambda i: (jnp.maximum(i - T, 0), 0)),
        out_shape=jax.ShapeDtypeStruct((Np, OD), jnp.float32),
        scratch_shapes=[pltpu.VMEM((K, D), jnp.float32),
                        pltpu.VMEM((K, OD), jnp.float32),
                        pltpu.VMEM((K, Np), jnp.float32)],
    )(x, eigenvectors, *smalls)

    return out[:N] if npad else out


# speedup vs baseline: 1.9003x; 1.9003x over previous
import jax
import jax.numpy as jnp
from jax.experimental import pallas as pl
from jax.experimental.pallas import tpu as pltpu

_TN = 10000


def _ln(t, g, b):
    mu = jnp.mean(t, axis=-1, keepdims=True)
    va = jnp.mean((t - mu) ** 2, axis=-1, keepdims=True)
    return (t - mu) * jax.lax.rsqrt(va + 1e-5) * g + b


def _dot(a, b, dims):
    return jax.lax.dot_general(a, b, (dims, ((), ())),
                               preferred_element_type=jnp.float32)


def _make_fused(T, tn):
    def fused(x_ref, e_ref, ev_ref, mrow_ref, mcol_ref,
              w1_ref, b1_ref, g1_ref, bb1_ref,
              w2_ref, b2_ref, g2_ref, bb2_ref,
              wq_ref, bq_ref, wk_ref, bk_ref, wv_ref, bv_ref,
              wo_ref, bo_ref, wf1_ref, bf1_ref, wf2_ref, bf2_ref,
              wp_ref, bp_ref, gp_ref, bbp_ref,
              out_ref, acc_ref, m_ref):
        i = pl.program_id(0)

        @pl.when(i < T)
        def _():
            part = _dot(e_ref[...], x_ref[...], ((0,), (0,)))

            @pl.when(i == 0)
            def _():
                acc_ref[...] = part

            @pl.when(i > 0)
            def _():
                acc_ref[...] = acc_ref[...] + part

        @pl.when(i == T - 1)
        def _():
            h = ev_ref[...] * w1_ref[...] + b1_ref[...]
            h = _ln(h, g1_ref[...], bb1_ref[...])
            h = jnp.maximum(h, 0.0)
            h = _dot(h, w2_ref[...], ((1,), (1,))) + b2_ref[...]
            h = _ln(h, g2_ref[...], bb2_ref[...])
            q = _dot(h, wq_ref[...], ((1,), (1,))) + bq_ref[...]
            k = _dot(h, wk_ref[...], ((1,), (1,))) + bk_ref[...]
            v = _dot(h, wv_ref[...], ((1,), (1,))) + bv_ref[...]
            mrow = mrow_ref[...]
            ctx_parts = []
            for hh in range(4):
                sl = slice(8 * hh, 8 * hh + 8)
                qh, kh, vh = q[:, sl], k[:, sl], v[:, sl]
                s = _dot(qh, kh, ((1,), (1,))) * (1.0 / jnp.sqrt(8.0))
                s = jnp.where(mrow == 0.0, -1e9, s)
                s = s - jnp.max(s, axis=-1, keepdims=True)
                e = jnp.exp(s)
                a = e / jnp.sum(e, axis=-1, keepdims=True)
                ctx_parts.append(_dot(a, vh, ((1,), (0,))))
            ctx = jnp.concatenate(ctx_parts, axis=1)
            ctx = _dot(ctx, wo_ref[...], ((1,), (1,))) + bo_ref[...]
            g = jnp.maximum(
                _dot(ctx, wf1_ref[...], ((1,), (1,))) + bf1_ref[...], 0.0)
            f = jnp.tanh(jnp.sum(g * wf2_ref[...], axis=1, keepdims=True)
                         + bf2_ref[...])
            f = f * mcol_ref[...]
            m_ref[...] = _dot(f * acc_ref[...], wp_ref[...], ((1,), (1,)))

        @pl.when(i >= T)
        def _():
            y = _dot(e_ref[...], m_ref[...], ((1,), (0,))) + bp_ref[...]
            out_ref[...] = _ln(y, gp_ref[...], bbp_ref[...])

    return fused


def kernel(x, eigenvectors, eigenvalues, W1, b1, g1, bb1, W2, b2, g2, bb2,
           Wq, bq, Wk, bk, Wv, bv, Wo, bo, Wf1, bf1, Wf2, bf2,
           Wp, bp, gp, bbp, eig_mask, batch):
    N, D = x.shape
    K = eigenvalues.shape[0]
    OD = Wp.shape[0]
    tn = _TN
    npad = (-N) % tn
    if npad:
        x = jnp.pad(x, ((0, npad), (0, 0)))
        eigenvectors = jnp.pad(eigenvectors, ((0, npad), (0, 0)))
    Np = N + npad
    T = Np // tn

    row = lambda a: a.reshape(1, -1).astype(jnp.float32)
    full = lambda shp: pl.BlockSpec(shp, lambda i: (0, 0))

    smalls = (
        eigenvalues.reshape(K, 1),
        row(eig_mask), eig_mask.astype(jnp.float32).reshape(K, 1),
        row(W1), row(b1), row(g1), row(bb1),
        W2, row(b2), row(g2), row(bb2),
        Wq, row(bq), Wk, row(bk), Wv, row(bv),
        Wo, row(bo), Wf1, row(bf1), row(Wf2), row(bf2),
        Wp, row(bp), row(gp), row(bbp),
    )
    small_specs = [full(a.shape) for a in smalls]

    out = pl.pallas_call(
        _make_fused(T, tn),
        grid=(2 * T,),
        in_specs=[pl.BlockSpec((tn, D), lambda i: (jnp.minimum(i, T - 1), 0)),
                  pl.BlockSpec((tn, K),
                               lambda i: (jnp.where(i < T, i, i - T), 0))]
                 + small_specs,
        out_specs=pl.BlockSpec((tn, OD), lambda i: (jnp.maximum(i - T, 0), 0)),
        out_shape=jax.ShapeDtypeStruct((Np, OD), jnp.float32),
        scratch_shapes=[pltpu.VMEM((K, D), jnp.float32),
                        pltpu.VMEM((K, OD), jnp.float32)],
    )(x, eigenvectors, *smalls)

    return out[:N] if npad else out
